# baseline (speedup 1.0000x reference)
import functools

import jax
import jax.numpy as jnp
from jax import lax
from jax.experimental import pallas as pl
from jax.experimental.pallas import tpu as pltpu
from jax.experimental.pallas import tpu_sc as plsc

E = 8
B = 4096
K = 2048
CM = 2048
N = 1024

R = 512
B_PAD = B + E * R
T = B_PAD // R

NC = 2
NS = 16
NW = NC * NS


def _route(idxs_b):
    counts = jnp.bincount(idxs_b, length=E).astype(jnp.int32)
    tiles_per = (counts + R - 1) // R
    padded_start = (jnp.cumsum(tiles_per) - tiles_per) * R
    start_sorted = jnp.cumsum(counts) - counts
    order = jnp.argsort(idxs_b).astype(jnp.int32)
    sorted_ids = idxs_b[order]
    p = jnp.arange(B, dtype=jnp.int32)
    rank = p - start_sorted[sorted_ids]
    dst_pos = (padded_start[sorted_ids] + rank).astype(jnp.int32)
    src = (jnp.arange(B_PAD, dtype=jnp.int32) % B).at[dst_pos].set(order)
    dest = jnp.zeros((B,), jnp.int32).at[order].set(dst_pos)
    tile_start = jnp.arange(T, dtype=jnp.int32) * R
    gid = jnp.sum(tile_start[:, None] >= padded_start[None, :], axis=1) - 1
    gid = jnp.clip(gid, 0, E - 1).astype(jnp.int32)
    return src, dest, gid


def _sc_gather(table, idx, chunk):
    M = idx.shape[0]
    D = table.shape[1]
    m_per_w = M // NW
    n = m_per_w // chunk
    NBUF = 3
    assert n >= NBUF and m_per_w % chunk == 0 and chunk % 8 == 0
    mesh = plsc.VectorSubcoreMesh(core_axis_name="c", subcore_axis_name="s")

    @functools.partial(
        pl.kernel,
        mesh=mesh,
        out_type=jax.ShapeDtypeStruct((M, D), table.dtype),
        scratch_types=[
            pltpu.VMEM((m_per_w,), jnp.int32),
            [pltpu.VMEM((chunk, D), table.dtype)] * NBUF,
            [pltpu.SemaphoreType.DMA] * NBUF,
            [pltpu.SemaphoreType.DMA] * NBUF,
        ],
    )
    def k(table_hbm, idx_hbm, out_hbm, idx_v, bufs, sg, sw):
        wid = lax.axis_index("s") * NC + lax.axis_index("c")
        base = wid * m_per_w
        pltpu.sync_copy(idx_hbm.at[pl.ds(base, m_per_w)], idx_v)

        def gcopy(j):
            b = j % NBUF
            return pltpu.make_async_copy(
                table_hbm.at[idx_v.at[pl.ds(j * chunk, chunk)]], bufs[b], sg[b])

        def wcopy(j):
            b = j % NBUF
            return pltpu.make_async_copy(
                bufs[b], out_hbm.at[pl.ds(base + j * chunk, chunk)], sw[b])

        for j in range(NBUF - 1):
            gcopy(j).start()
        for j in range(n):
            nj = j + NBUF - 1
            if nj < n:
                if nj >= NBUF:
                    wcopy(nj - NBUF).wait()
                gcopy(nj).start()
            gcopy(j).wait()
            wcopy(j).start()
        for j in range(n - NBUF, n):
            wcopy(j).wait()

    return k(table, idx)


def _mlp_body(gid_ref, x_ref, w1_ref, b1_ref, w2_ref, b2_ref, o_ref):
    x = x_ref[...].astype(jnp.bfloat16)
    h = jnp.maximum(
        jnp.dot(x, w1_ref[0], preferred_element_type=jnp.float32)
        + b1_ref[0],
        0.0,
    ).astype(jnp.bfloat16)
    o_ref[...] = (
        jnp.dot(h, w2_ref[0], preferred_element_type=jnp.float32)
        + b2_ref[0]
    )


def _mlp(xs, W1, b1, W2, b2, gid):
    grid_spec = pltpu.PrefetchScalarGridSpec(
        num_scalar_prefetch=1,
        grid=(T,),
        in_specs=[
            pl.BlockSpec((R, K), lambda t, g: (t, 0)),
            pl.BlockSpec((1, K, CM), lambda t, g: (g[t], 0, 0)),
            pl.BlockSpec((1, 1, CM), lambda t, g: (g[t], 0, 0)),
            pl.BlockSpec((1, CM, N), lambda t, g: (g[t], 0, 0)),
            pl.BlockSpec((1, 1, N), lambda t, g: (g[t], 0, 0)),
        ],
        out_specs=pl.BlockSpec((R, N), lambda t, g: (t, 0)),
    )
    return pl.pallas_call(
        _mlp_body,
        grid_spec=grid_spec,
        out_shape=jax.ShapeDtypeStruct((B_PAD, N), jnp.float32),
        compiler_params=pltpu.CompilerParams(
            dimension_semantics=("arbitrary",),
        ),
    )(gid, xs, W1.astype(jnp.bfloat16), b1.reshape(E, 1, CM),
      W2.astype(jnp.bfloat16), b2.reshape(E, 1, N))


def kernel(xis, idxs_b, W1, b1, W2, b2):
    src, dest, gid = _route(idxs_b)
    xs = _sc_gather(xis, src, 16)
    ys = _mlp(xs, W1, b1, W2, b2, gid)
    return _sc_gather(ys, dest, 32)


# speedup vs baseline: 1.1820x; 1.1820x over previous
import functools

import jax
import jax.numpy as jnp
from jax import lax
from jax.experimental import pallas as pl
from jax.experimental.pallas import tpu as pltpu
from jax.experimental.pallas import tpu_sc as plsc

E = 8
B = 4096
K = 2048
CM = 2048
N = 1024

R = 512
TCM = 1024
B_PAD = B + E * R
T = B_PAD // R
CMT = CM // TCM

NC = 2
NS = 16
NW = NC * NS


def _route(idxs_b):
    counts = jnp.bincount(idxs_b, length=E).astype(jnp.int32)
    tiles_per = (counts + R - 1) // R
    padded_start = (jnp.cumsum(tiles_per) - tiles_per) * R
    start_sorted = jnp.cumsum(counts) - counts
    order = jnp.argsort(idxs_b).astype(jnp.int32)
    sorted_ids = idxs_b[order]
    p = jnp.arange(B, dtype=jnp.int32)
    rank = p - start_sorted[sorted_ids]
    dst_pos = (padded_start[sorted_ids] + rank).astype(jnp.int32)
    src = (jnp.arange(B_PAD, dtype=jnp.int32) % B).at[dst_pos].set(order)
    dest = jnp.zeros((B,), jnp.int32).at[order].set(dst_pos)
    tile_start = jnp.arange(T, dtype=jnp.int32) * R
    gid = jnp.sum(tile_start[:, None] >= padded_start[None, :], axis=1) - 1
    gid = jnp.clip(gid, 0, E - 1).astype(jnp.int32)
    return src, dest, gid


def _sc_gather(table, idx, chunk):
    M = idx.shape[0]
    D = table.shape[1]
    m_per_w = M // NW
    n = m_per_w // chunk
    NBUF = 3
    assert n >= NBUF and m_per_w % chunk == 0 and chunk % 8 == 0
    mesh = plsc.VectorSubcoreMesh(core_axis_name="c", subcore_axis_name="s")

    @functools.partial(
        pl.kernel,
        mesh=mesh,
        out_type=jax.ShapeDtypeStruct((M, D), table.dtype),
        scratch_types=[
            pltpu.VMEM((m_per_w,), jnp.int32),
            [pltpu.VMEM((chunk, D), table.dtype)] * NBUF,
            [pltpu.SemaphoreType.DMA] * NBUF,
            [pltpu.SemaphoreType.DMA] * NBUF,
        ],
    )
    def k(table_hbm, idx_hbm, out_hbm, idx_v, bufs, sg, sw):
        wid = lax.axis_index("s") * NC + lax.axis_index("c")
        base = wid * m_per_w
        pltpu.sync_copy(idx_hbm.at[pl.ds(base, m_per_w)], idx_v)

        def gcopy(j):
            b = j % NBUF
            return pltpu.make_async_copy(
                table_hbm.at[idx_v.at[pl.ds(j * chunk, chunk)]], bufs[b], sg[b])

        def wcopy(j):
            b = j % NBUF
            return pltpu.make_async_copy(
                bufs[b], out_hbm.at[pl.ds(base + j * chunk, chunk)], sw[b])

        for j in range(NBUF - 1):
            gcopy(j).start()
        for j in range(n):
            nj = j + NBUF - 1
            if nj < n:
                if nj >= NBUF:
                    wcopy(nj - NBUF).wait()
                gcopy(nj).start()
            gcopy(j).wait()
            wcopy(j).start()
        for j in range(n - NBUF, n):
            wcopy(j).wait()

    return k(table, idx)


def _mlp_body(gid_ref, x_ref, w1_ref, b1_ref, w2_ref, b2_ref, o_ref):
    c = pl.program_id(1)
    x = x_ref[...].astype(jnp.bfloat16)
    h = jnp.maximum(
        jnp.dot(x, w1_ref[0].astype(jnp.bfloat16),
                preferred_element_type=jnp.float32)
        + b1_ref[0],
        0.0,
    ).astype(jnp.bfloat16)
    contrib = jnp.dot(h, w2_ref[0].astype(jnp.bfloat16),
                      preferred_element_type=jnp.float32)

    @pl.when(c == 0)
    def _():
        o_ref[...] = contrib + b2_ref[0]

    @pl.when(c != 0)
    def _():
        o_ref[...] += contrib


def _mlp(xs, W1, b1, W2, b2, gid):
    grid_spec = pltpu.PrefetchScalarGridSpec(
        num_scalar_prefetch=1,
        grid=(T, CMT),
        in_specs=[
            pl.BlockSpec((R, K), lambda t, c, g: (t, 0)),
            pl.BlockSpec((1, K, TCM), lambda t, c, g: (g[t], 0, c)),
            pl.BlockSpec((1, 1, TCM), lambda t, c, g: (g[t], 0, c)),
            pl.BlockSpec((1, TCM, N), lambda t, c, g: (g[t], c, 0)),
            pl.BlockSpec((1, 1, N), lambda t, c, g: (g[t], 0, 0)),
        ],
        out_specs=pl.BlockSpec((R, N), lambda t, c, g: (t, 0)),
    )
    return pl.pallas_call(
        _mlp_body,
        grid_spec=grid_spec,
        out_shape=jax.ShapeDtypeStruct((B_PAD, N), jnp.float32),
        compiler_params=pltpu.CompilerParams(
            dimension_semantics=("arbitrary", "arbitrary"),
        ),
    )(gid, xs, W1, b1.reshape(E, 1, CM), W2, b2.reshape(E, 1, N))


def kernel(xis, idxs_b, W1, b1, W2, b2):
    src, dest, gid = _route(idxs_b)
    xs = _sc_gather(xis, src, 16)
    ys = _mlp(xs, W1, b1, W2, b2, gid)
    return _sc_gather(ys, dest, 32)


# speedup vs baseline: 1.3027x; 1.1021x over previous
import functools

import jax
import jax.numpy as jnp
from jax import lax
from jax.experimental import pallas as pl
from jax.experimental.pallas import tpu as pltpu
from jax.experimental.pallas import tpu_sc as plsc

E = 8
B = 4096
K = 2048
CM = 2048
N = 1024

R = 512
TCM = 1024
B_PAD = B + E * R
T = B_PAD // R
CMT = CM // TCM

NC = 2
NS = 16
NW = NC * NS


def _route(idxs_b):
    counts = jnp.bincount(idxs_b, length=E).astype(jnp.int32)
    tiles_per = (counts + R - 1) // R
    padded_start = (jnp.cumsum(tiles_per) - tiles_per) * R
    start_sorted = jnp.cumsum(counts) - counts
    order = jnp.argsort(idxs_b).astype(jnp.int32)
    sorted_ids = idxs_b[order]
    p = jnp.arange(B, dtype=jnp.int32)
    rank = p - start_sorted[sorted_ids]
    dst_pos = (padded_start[sorted_ids] + rank).astype(jnp.int32)
    src = (jnp.arange(B_PAD, dtype=jnp.int32) % B).at[dst_pos].set(order)
    dest = jnp.zeros((B,), jnp.int32).at[order].set(dst_pos)
    tile_start = jnp.arange(T, dtype=jnp.int32) * R
    gid = jnp.sum(tile_start[:, None] >= padded_start[None, :], axis=1) - 1
    gid = jnp.clip(gid, 0, E - 1).astype(jnp.int32)
    return src, dest, gid


def _sc_gather(table, idx, chunk):
    M = idx.shape[0]
    D = table.shape[1]
    m_per_w = M // NW
    n = m_per_w // chunk
    NBUF = 3
    assert n >= NBUF and m_per_w % chunk == 0 and chunk % 8 == 0
    mesh = plsc.VectorSubcoreMesh(core_axis_name="c", subcore_axis_name="s")

    @functools.partial(
        pl.kernel,
        mesh=mesh,
        out_type=jax.ShapeDtypeStruct((M, D), table.dtype),
        scratch_types=[
            pltpu.VMEM((m_per_w,), jnp.int32),
            [pltpu.VMEM((chunk, D), table.dtype)] * NBUF,
            [pltpu.SemaphoreType.DMA] * NBUF,
            [pltpu.SemaphoreType.DMA] * NBUF,
        ],
    )
    def k(table_hbm, idx_hbm, out_hbm, idx_v, bufs, sg, sw):
        wid = lax.axis_index("s") * NC + lax.axis_index("c")
        base = wid * m_per_w
        pltpu.sync_copy(idx_hbm.at[pl.ds(base, m_per_w)], idx_v)

        def gcopy(j):
            b = j % NBUF
            return pltpu.make_async_copy(
                table_hbm.at[idx_v.at[pl.ds(j * chunk, chunk)]], bufs[b], sg[b])

        def wcopy(j):
            b = j % NBUF
            return pltpu.make_async_copy(
                bufs[b], out_hbm.at[pl.ds(base + j * chunk, chunk)], sw[b])

        for j in range(NBUF - 1):
            gcopy(j).start()
        for j in range(n):
            nj = j + NBUF - 1
            if nj < n:
                if nj >= NBUF:
                    wcopy(nj - NBUF).wait()
                gcopy(nj).start()
            gcopy(j).wait()
            wcopy(j).start()
        for j in range(n - NBUF, n):
            wcopy(j).wait()

    return k(table, idx)


def _mlp_body(gid_ref, x_ref, w1_ref, b1_ref, w2_ref, b2_ref, o_ref):
    x = x_ref[...].astype(jnp.bfloat16)
    h = jnp.maximum(
        jnp.dot(x, w1_ref[0].astype(jnp.bfloat16),
                preferred_element_type=jnp.float32)
        + b1_ref[0],
        0.0,
    ).astype(jnp.bfloat16)
    o_ref[...] = (
        jnp.dot(h, w2_ref[0].astype(jnp.bfloat16),
                preferred_element_type=jnp.float32)
        + b2_ref[0]
    )


def _mlp(xs, W1, b1, W2, b2, gid):
    grid_spec = pltpu.PrefetchScalarGridSpec(
        num_scalar_prefetch=1,
        grid=(T,),
        in_specs=[
            pl.BlockSpec((R, K), lambda t, g: (t, 0)),
            pl.BlockSpec((1, K, CM), lambda t, g: (g[t], 0, 0)),
            pl.BlockSpec((1, 1, CM), lambda t, g: (g[t], 0, 0)),
            pl.BlockSpec((1, CM, N), lambda t, g: (g[t], 0, 0)),
            pl.BlockSpec((1, 1, N), lambda t, g: (g[t], 0, 0)),
        ],
        out_specs=pl.BlockSpec((R, N), lambda t, g: (t, 0)),
    )
    return pl.pallas_call(
        _mlp_body,
        grid_spec=grid_spec,
        out_shape=jax.ShapeDtypeStruct((B_PAD, N), jnp.float32),
        compiler_params=pltpu.CompilerParams(
            dimension_semantics=("arbitrary",),
            vmem_limit_bytes=100 * 1024 * 1024,
        ),
    )(gid, xs, W1, b1.reshape(E, 1, CM), W2, b2.reshape(E, 1, N))


def kernel(xis, idxs_b, W1, b1, W2, b2):
    src, dest, gid = _route(idxs_b)
    xs = _sc_gather(xis, src, 16)
    ys = _mlp(xs, W1, b1, W2, b2, gid)
    return _sc_gather(ys, dest, 32)
